# Initial kernel scaffold; baseline (speedup 1.0000x reference)
import jax
import jax.numpy as jnp
from jax.experimental import pallas as pl


def kernel(h, mps, mp_edge, gcn_W, gcn_bfc, gcn_bias, gcn_a, att_W, att_b, att_v):
    raise NotImplementedError("write your pallas kernel here")


# speedup vs baseline: 1.4208x; 1.4208x over previous
import jax
import jax.numpy as jnp
from jax.experimental import pallas as pl
from jax.experimental.pallas import tpu as pltpu

HID = 256
N = 4096
BM = 512
NB = N // BM


def _mp_body(h_ref, adj_ref, wt_ref, gp_ref, awt_ref, ab_ref, e_ref, cs_ref,
             sf_ref):
    i = pl.program_id(1)

    @pl.when(i == 0)
    def _():
        sf_ref[...] = (
            jnp.dot(h_ref[...], wt_ref[0], preferred_element_type=jnp.float32)
            + gp_ref[0, 0, :][None, :])

    o = (jnp.dot(adj_ref[0], sf_ref[...], preferred_element_type=jnp.float32)
         + gp_ref[0, 1, :][None, :])
    a = gp_ref[0, 2, :][None, :]
    e = jnp.where(o > 0, o, a * o)
    e_ref[0] = e
    t = jnp.tanh(
        jnp.dot(e, awt_ref[0], preferred_element_type=jnp.float32)
        + ab_ref[0, 0, :][None, :])
    part = jnp.sum(t, axis=0, keepdims=True)

    @pl.when(i == 0)
    def _():
        cs_ref[0] = part

    @pl.when(i > 0)
    def _():
        cs_ref[0] = cs_ref[0] + part


def _mix_body(ea_ref, eb_ref, cs_ref, av_ref, o_ref):
    p = pl.program_id(0)
    av = av_ref[0, 0, :]
    l0 = jnp.sum(av * cs_ref[2 * p, 0, :]) * (1.0 / N)
    l1 = jnp.sum(av * cs_ref[2 * p + 1, 0, :]) * (1.0 / N)
    m = jnp.maximum(l0, l1)
    x0 = jnp.exp(l0 - m)
    x1 = jnp.exp(l1 - m)
    b0 = x0 / (x0 + x1)
    b1 = x1 / (x0 + x1)
    o_ref[...] = b0 * ea_ref[0] + b1 * eb_ref[0]


def kernel(h, mps, mp_edge, gcn_W, gcn_bfc, gcn_bias, gcn_a, att_W, att_b,
           att_v):
    del mp_edge
    gwt = jnp.swapaxes(gcn_W, 1, 2)
    awt = jnp.swapaxes(att_W, 1, 2)
    gp = jnp.stack(
        [gcn_bfc, gcn_bias, jnp.broadcast_to(gcn_a[:, None], (4, HID))],
        axis=1)
    ab = att_b[:, None, :]
    av = att_v[:, None, :]

    e, cs = pl.pallas_call(
        _mp_body,
        grid=(4, NB),
        in_specs=[
            pl.BlockSpec((N, HID), lambda k, i: (k // 2, 0)),
            pl.BlockSpec((1, BM, N), lambda k, i: (k, i, 0)),
            pl.BlockSpec((1, HID, HID), lambda k, i: (k, 0, 0)),
            pl.BlockSpec((1, 3, HID), lambda k, i: (k, 0, 0)),
            pl.BlockSpec((1, HID, HID), lambda k, i: (k // 2, 0, 0)),
            pl.BlockSpec((1, 1, HID), lambda k, i: (k // 2, 0, 0)),
        ],
        out_specs=[
            pl.BlockSpec((1, BM, HID), lambda k, i: (k, i, 0)),
            pl.BlockSpec((1, 1, HID), lambda k, i: (k, 0, 0)),
        ],
        out_shape=[
            jax.ShapeDtypeStruct((4, N, HID), jnp.float32),
            jax.ShapeDtypeStruct((4, 1, HID), jnp.float32),
        ],
        scratch_shapes=[pltpu.VMEM((N, HID), jnp.float32)],
    )(h, mps, gwt, gp, awt, ab)

    z = pl.pallas_call(
        _mix_body,
        grid=(2, NB),
        in_specs=[
            pl.BlockSpec((1, BM, HID), lambda p, i: (2 * p, i, 0)),
            pl.BlockSpec((1, BM, HID), lambda p, i: (2 * p + 1, i, 0)),
            pl.BlockSpec((4, 1, HID), lambda p, i: (0, 0, 0)),
            pl.BlockSpec((1, 1, HID), lambda p, i: (p, 0, 0)),
        ],
        out_specs=pl.BlockSpec((BM, HID), lambda p, i: (p * NB + i, 0)),
        out_shape=jax.ShapeDtypeStruct((2 * N, HID), jnp.float32),
    )(e, e, cs, av)
    return z


# speedup vs baseline: 1.4236x; 1.0020x over previous
import jax
import jax.numpy as jnp
from jax.experimental import pallas as pl
from jax.experimental.pallas import tpu as pltpu

HID = 256
N = 4096
BM = 512
NB = N // BM


def _mp_body(h_ref, adj_ref, wt_ref, gp_ref, awt_ref, ab_ref, e_ref, cs_ref,
             sf_ref):
    i = pl.program_id(1)

    @pl.when(i == 0)
    def _():
        sf_ref[...] = (
            jnp.dot(h_ref[...], wt_ref[0], preferred_element_type=jnp.float32)
            + gp_ref[0, 0, :][None, :]).astype(jnp.bfloat16)

    o = (jnp.dot(adj_ref[0].astype(jnp.bfloat16), sf_ref[...],
                 preferred_element_type=jnp.float32)
         + gp_ref[0, 1, :][None, :])
    a = gp_ref[0, 2, :][None, :]
    e = jnp.where(o > 0, o, a * o)
    e_ref[0] = e
    t = jnp.tanh(
        jnp.dot(e, awt_ref[0], preferred_element_type=jnp.float32)
        + ab_ref[0, 0, :][None, :])
    part = jnp.sum(t, axis=0, keepdims=True)

    @pl.when(i == 0)
    def _():
        cs_ref[0] = part

    @pl.when(i > 0)
    def _():
        cs_ref[0] = cs_ref[0] + part


def _mix_body(ea_ref, eb_ref, cs_ref, av_ref, o_ref):
    p = pl.program_id(0)
    av = av_ref[0, 0, :]
    l0 = jnp.sum(av * cs_ref[2 * p, 0, :]) * (1.0 / N)
    l1 = jnp.sum(av * cs_ref[2 * p + 1, 0, :]) * (1.0 / N)
    m = jnp.maximum(l0, l1)
    x0 = jnp.exp(l0 - m)
    x1 = jnp.exp(l1 - m)
    b0 = x0 / (x0 + x1)
    b1 = x1 / (x0 + x1)
    o_ref[...] = b0 * ea_ref[0] + b1 * eb_ref[0]


def kernel(h, mps, mp_edge, gcn_W, gcn_bfc, gcn_bias, gcn_a, att_W, att_b,
           att_v):
    del mp_edge
    gwt = jnp.swapaxes(gcn_W, 1, 2)
    awt = jnp.swapaxes(att_W, 1, 2)
    gp = jnp.stack(
        [gcn_bfc, gcn_bias, jnp.broadcast_to(gcn_a[:, None], (4, HID))],
        axis=1)
    ab = att_b[:, None, :]
    av = att_v[:, None, :]

    e, cs = pl.pallas_call(
        _mp_body,
        grid=(4, NB),
        in_specs=[
            pl.BlockSpec((N, HID), lambda k, i: (k // 2, 0)),
            pl.BlockSpec((1, BM, N), lambda k, i: (k, i, 0)),
            pl.BlockSpec((1, HID, HID), lambda k, i: (k, 0, 0)),
            pl.BlockSpec((1, 3, HID), lambda k, i: (k, 0, 0)),
            pl.BlockSpec((1, HID, HID), lambda k, i: (k // 2, 0, 0)),
            pl.BlockSpec((1, 1, HID), lambda k, i: (k // 2, 0, 0)),
        ],
        out_specs=[
            pl.BlockSpec((1, BM, HID), lambda k, i: (k, i, 0)),
            pl.BlockSpec((1, 1, HID), lambda k, i: (k, 0, 0)),
        ],
        out_shape=[
            jax.ShapeDtypeStruct((4, N, HID), jnp.float32),
            jax.ShapeDtypeStruct((4, 1, HID), jnp.float32),
        ],
        scratch_shapes=[pltpu.VMEM((N, HID), jnp.bfloat16)],
    )(h, mps, gwt, gp, awt, ab)

    z = pl.pallas_call(
        _mix_body,
        grid=(2, NB),
        in_specs=[
            pl.BlockSpec((1, BM, HID), lambda p, i: (2 * p, i, 0)),
            pl.BlockSpec((1, BM, HID), lambda p, i: (2 * p + 1, i, 0)),
            pl.BlockSpec((4, 1, HID), lambda p, i: (0, 0, 0)),
            pl.BlockSpec((1, 1, HID), lambda p, i: (p, 0, 0)),
        ],
        out_specs=pl.BlockSpec((BM, HID), lambda p, i: (p * NB + i, 0)),
        out_shape=jax.ShapeDtypeStruct((2 * N, HID), jnp.float32),
    )(e, e, cs, av)
    return z


# speedup vs baseline: 1.4638x; 1.0282x over previous
import jax
import jax.numpy as jnp
from jax.experimental import pallas as pl
from jax.experimental.pallas import tpu as pltpu

HID = 256
N = 4096
BM = 512
NB = N // BM


def _mp_body(h_ref, adj_ref, wt_ref, gp_ref, awt_ref, ab_ref, e_ref, cs_ref,
             sf_ref):
    i = pl.program_id(1)

    @pl.when(i == 0)
    def _():
        sf_ref[...] = (
            jnp.dot(h_ref[...].astype(jnp.bfloat16),
                    wt_ref[0].astype(jnp.bfloat16),
                    preferred_element_type=jnp.float32)
            + gp_ref[0, 0, :][None, :]).astype(jnp.bfloat16)

    o = (jnp.dot(adj_ref[0].astype(jnp.bfloat16), sf_ref[...],
                 preferred_element_type=jnp.float32)
         + gp_ref[0, 1, :][None, :])
    a = gp_ref[0, 2, :][None, :]
    e = jnp.where(o > 0, o, a * o).astype(jnp.bfloat16)
    e_ref[0] = e
    t = jnp.tanh(
        jnp.dot(e, awt_ref[0].astype(jnp.bfloat16),
                preferred_element_type=jnp.float32)
        + ab_ref[0, 0, :][None, :])
    part = jnp.sum(t, axis=0, keepdims=True)

    @pl.when(i == 0)
    def _():
        cs_ref[0] = part

    @pl.when(i > 0)
    def _():
        cs_ref[0] = cs_ref[0] + part


def _mix_body(ea_ref, eb_ref, cs_ref, av_ref, o_ref):
    p = pl.program_id(0)
    av = av_ref[0, 0, :]
    l0 = jnp.sum(av * cs_ref[2 * p, 0, :]) * (1.0 / N)
    l1 = jnp.sum(av * cs_ref[2 * p + 1, 0, :]) * (1.0 / N)
    m = jnp.maximum(l0, l1)
    x0 = jnp.exp(l0 - m)
    x1 = jnp.exp(l1 - m)
    b0 = x0 / (x0 + x1)
    b1 = x1 / (x0 + x1)
    o_ref[...] = (b0 * ea_ref[0].astype(jnp.float32)
                  + b1 * eb_ref[0].astype(jnp.float32))


def kernel(h, mps, mp_edge, gcn_W, gcn_bfc, gcn_bias, gcn_a, att_W, att_b,
           att_v):
    del mp_edge
    gwt = jnp.swapaxes(gcn_W, 1, 2)
    awt = jnp.swapaxes(att_W, 1, 2)
    gp = jnp.stack(
        [gcn_bfc, gcn_bias, jnp.broadcast_to(gcn_a[:, None], (4, HID))],
        axis=1)
    ab = att_b[:, None, :]
    av = att_v[:, None, :]

    e, cs = pl.pallas_call(
        _mp_body,
        grid=(4, NB),
        in_specs=[
            pl.BlockSpec((N, HID), lambda k, i: (k // 2, 0)),
            pl.BlockSpec((1, BM, N), lambda k, i: (k, i, 0)),
            pl.BlockSpec((1, HID, HID), lambda k, i: (k, 0, 0)),
            pl.BlockSpec((1, 3, HID), lambda k, i: (k, 0, 0)),
            pl.BlockSpec((1, HID, HID), lambda k, i: (k // 2, 0, 0)),
            pl.BlockSpec((1, 1, HID), lambda k, i: (k // 2, 0, 0)),
        ],
        out_specs=[
            pl.BlockSpec((1, BM, HID), lambda k, i: (k, i, 0)),
            pl.BlockSpec((1, 1, HID), lambda k, i: (k, 0, 0)),
        ],
        out_shape=[
            jax.ShapeDtypeStruct((4, N, HID), jnp.bfloat16),
            jax.ShapeDtypeStruct((4, 1, HID), jnp.float32),
        ],
        scratch_shapes=[pltpu.VMEM((N, HID), jnp.bfloat16)],
    )(h, mps, gwt, gp, awt, ab)

    z = pl.pallas_call(
        _mix_body,
        grid=(2, NB),
        in_specs=[
            pl.BlockSpec((1, BM, HID), lambda p, i: (2 * p, i, 0)),
            pl.BlockSpec((1, BM, HID), lambda p, i: (2 * p + 1, i, 0)),
            pl.BlockSpec((4, 1, HID), lambda p, i: (0, 0, 0)),
            pl.BlockSpec((1, 1, HID), lambda p, i: (p, 0, 0)),
        ],
        out_specs=pl.BlockSpec((BM, HID), lambda p, i: (p * NB + i, 0)),
        out_shape=jax.ShapeDtypeStruct((2 * N, HID), jnp.float32),
    )(e, e, cs, av)
    return z


# speedup vs baseline: 1.5681x; 1.0713x over previous
import jax
import jax.numpy as jnp
from jax.experimental import pallas as pl
from jax.experimental.pallas import tpu as pltpu

HID = 256
N = 4096
BM = 1024
NB = N // BM


def _mp_body(h_ref, adj_ref, wt_ref, gp_ref, awt_ref, ab_ref, e_ref, cs_ref,
             sf_ref):
    i = pl.program_id(1)

    @pl.when(i == 0)
    def _():
        sf_ref[...] = (
            jnp.dot(h_ref[...].astype(jnp.bfloat16),
                    wt_ref[0].astype(jnp.bfloat16),
                    preferred_element_type=jnp.float32)
            + gp_ref[0, 0, :][None, :]).astype(jnp.bfloat16)

    o = (jnp.dot(adj_ref[0].astype(jnp.bfloat16), sf_ref[...],
                 preferred_element_type=jnp.float32)
         + gp_ref[0, 1, :][None, :])
    a = gp_ref[0, 2, :][None, :]
    e = jnp.where(o > 0, o, a * o).astype(jnp.bfloat16)
    e_ref[0] = e
    t = jnp.tanh(
        jnp.dot(e, awt_ref[0].astype(jnp.bfloat16),
                preferred_element_type=jnp.float32)
        + ab_ref[0, 0, :][None, :])
    part = jnp.sum(t, axis=0, keepdims=True)

    @pl.when(i == 0)
    def _():
        cs_ref[0] = part

    @pl.when(i > 0)
    def _():
        cs_ref[0] = cs_ref[0] + part


def _mix_body(ea_ref, eb_ref, cs_ref, av_ref, o_ref):
    p = pl.program_id(0)
    av = av_ref[0, 0, :]
    l0 = jnp.sum(av * cs_ref[2 * p, 0, :]) * (1.0 / N)
    l1 = jnp.sum(av * cs_ref[2 * p + 1, 0, :]) * (1.0 / N)
    m = jnp.maximum(l0, l1)
    x0 = jnp.exp(l0 - m)
    x1 = jnp.exp(l1 - m)
    b0 = x0 / (x0 + x1)
    b1 = x1 / (x0 + x1)
    o_ref[...] = (b0 * ea_ref[0].astype(jnp.float32)
                  + b1 * eb_ref[0].astype(jnp.float32))


def kernel(h, mps, mp_edge, gcn_W, gcn_bfc, gcn_bias, gcn_a, att_W, att_b,
           att_v):
    del mp_edge
    gwt = jnp.swapaxes(gcn_W, 1, 2)
    awt = jnp.swapaxes(att_W, 1, 2)
    gp = jnp.stack(
        [gcn_bfc, gcn_bias, jnp.broadcast_to(gcn_a[:, None], (4, HID))],
        axis=1)
    ab = att_b[:, None, :]
    av = att_v[:, None, :]

    e, cs = pl.pallas_call(
        _mp_body,
        grid=(4, NB),
        in_specs=[
            pl.BlockSpec((N, HID), lambda k, i: (k // 2, 0)),
            pl.BlockSpec((1, BM, N), lambda k, i: (k, i, 0)),
            pl.BlockSpec((1, HID, HID), lambda k, i: (k, 0, 0)),
            pl.BlockSpec((1, 3, HID), lambda k, i: (k, 0, 0)),
            pl.BlockSpec((1, HID, HID), lambda k, i: (k // 2, 0, 0)),
            pl.BlockSpec((1, 1, HID), lambda k, i: (k // 2, 0, 0)),
        ],
        out_specs=[
            pl.BlockSpec((1, BM, HID), lambda k, i: (k, i, 0)),
            pl.BlockSpec((1, 1, HID), lambda k, i: (k, 0, 0)),
        ],
        out_shape=[
            jax.ShapeDtypeStruct((4, N, HID), jnp.bfloat16),
            jax.ShapeDtypeStruct((4, 1, HID), jnp.float32),
        ],
        scratch_shapes=[pltpu.VMEM((N, HID), jnp.bfloat16)],
    )(h, mps, gwt, gp, awt, ab)

    z = pl.pallas_call(
        _mix_body,
        grid=(2, NB),
        in_specs=[
            pl.BlockSpec((1, BM, HID), lambda p, i: (2 * p, i, 0)),
            pl.BlockSpec((1, BM, HID), lambda p, i: (2 * p + 1, i, 0)),
            pl.BlockSpec((4, 1, HID), lambda p, i: (0, 0, 0)),
            pl.BlockSpec((1, 1, HID), lambda p, i: (p, 0, 0)),
        ],
        out_specs=pl.BlockSpec((BM, HID), lambda p, i: (p * NB + i, 0)),
        out_shape=jax.ShapeDtypeStruct((2 * N, HID), jnp.float32),
    )(e, e, cs, av)
    return z


# speedup vs baseline: 1.6198x; 1.0330x over previous
import jax
import jax.numpy as jnp
from jax.experimental import pallas as pl
from jax.experimental.pallas import tpu as pltpu

HID = 256
N = 4096
BM = 1024
NB = N // BM


def _mix(cs_ref, av_ref, es_ref, z_ref, pair, i):
    c0, c1 = 2 * pair, 2 * pair + 1
    av = av_ref[pair, 0, :]
    l0 = jnp.sum(av * cs_ref[c0, 0, :]) * (1.0 / N)
    l1 = jnp.sum(av * cs_ref[c1, 0, :]) * (1.0 / N)
    m = jnp.maximum(l0, l1)
    x0 = jnp.exp(l0 - m)
    x1 = jnp.exp(l1 - m)
    b0 = x0 / (x0 + x1)
    b1 = x1 / (x0 + x1)
    z_ref[...] = (b0 * es_ref[c0, i].astype(jnp.float32)
                  + b1 * es_ref[c1, i].astype(jnp.float32))


def _body(h_ref, adj_ref, wt_ref, gp_ref, awt_ref, ab_ref, av_ref, z_ref,
          sf_ref, es_ref, cs_ref):
    k = pl.program_id(0)
    i = pl.program_id(1)

    @pl.when(k < 4)
    def _():
        @pl.when(i == 0)
        def _():
            sf_ref[...] = (
                jnp.dot(h_ref[...].astype(jnp.bfloat16),
                        wt_ref[0].astype(jnp.bfloat16),
                        preferred_element_type=jnp.float32)
                + gp_ref[0, 0, :][None, :]).astype(jnp.bfloat16)

        o = (jnp.dot(adj_ref[0].astype(jnp.bfloat16), sf_ref[...],
                     preferred_element_type=jnp.float32)
             + gp_ref[0, 1, :][None, :])
        a = gp_ref[0, 2, :][None, :]
        e = jnp.where(o > 0, o, a * o).astype(jnp.bfloat16)
        es_ref[k, i] = e
        t = jnp.tanh(
            jnp.dot(e, awt_ref[0].astype(jnp.bfloat16),
                    preferred_element_type=jnp.float32)
            + ab_ref[k // 2, 0, :][None, :])
        part = jnp.sum(t, axis=0, keepdims=True)

        @pl.when(i == 0)
        def _():
            cs_ref[k] = part

        @pl.when(i > 0)
        def _():
            cs_ref[k] = cs_ref[k] + part

    @pl.when(k == 2)
    def _():
        _mix(cs_ref, av_ref, es_ref, z_ref, 0, i)

    @pl.when(k == 4)
    def _():
        _mix(cs_ref, av_ref, es_ref, z_ref, 1, i)


def kernel(h, mps, mp_edge, gcn_W, gcn_bfc, gcn_bias, gcn_a, att_W, att_b,
           att_v):
    del mp_edge
    gwt = jnp.swapaxes(gcn_W, 1, 2)
    awt = jnp.swapaxes(att_W, 1, 2)
    gp = jnp.stack(
        [gcn_bfc, gcn_bias, jnp.broadcast_to(gcn_a[:, None], (4, HID))],
        axis=1)
    ab = att_b[:, None, :]
    av = att_v[:, None, :]

    def zmap(k, i):
        blk = jnp.where(k < 2, 0,
                        jnp.where(k == 2, i,
                                  jnp.where(k == 3, NB - 1, NB + i)))
        return (blk, 0)

    z = pl.pallas_call(
        _body,
        grid=(5, NB),
        in_specs=[
            pl.BlockSpec((N, HID), lambda k, i: (jnp.minimum(k, 3) // 2, 0)),
            pl.BlockSpec(
                (1, BM, N),
                lambda k, i: (jnp.minimum(k, 3),
                              jnp.where(k < 4, i, NB - 1), 0)),
            pl.BlockSpec((1, HID, HID), lambda k, i: (jnp.minimum(k, 3), 0, 0)),
            pl.BlockSpec((1, 3, HID), lambda k, i: (jnp.minimum(k, 3), 0, 0)),
            pl.BlockSpec((1, HID, HID),
                         lambda k, i: (jnp.minimum(k, 3) // 2, 0, 0)),
            pl.BlockSpec((2, 1, HID), lambda k, i: (0, 0, 0)),
            pl.BlockSpec((2, 1, HID), lambda k, i: (0, 0, 0)),
        ],
        out_specs=pl.BlockSpec((BM, HID), zmap),
        out_shape=jax.ShapeDtypeStruct((2 * N, HID), jnp.float32),
        scratch_shapes=[
            pltpu.VMEM((N, HID), jnp.bfloat16),
            pltpu.VMEM((4, NB, BM, HID), jnp.bfloat16),
            pltpu.VMEM((4, 1, HID), jnp.float32),
        ],
    )(h, mps, gwt, gp, awt, ab, av)
    return z


# speedup vs baseline: 1.6714x; 1.0319x over previous
import jax
import jax.numpy as jnp
from jax.experimental import pallas as pl
from jax.experimental.pallas import tpu as pltpu

HID = 256
N = 4096
BM = 1024
NB = N // BM


def _mix(cs_ref, av_ref, es_ref, z_ref, pair, i):
    c0, c1 = 2 * pair, 2 * pair + 1
    av = av_ref[pair, 0, :]
    l0 = jnp.sum(av * cs_ref[c0, 0, :]) * (1.0 / N)
    l1 = jnp.sum(av * cs_ref[c1, 0, :]) * (1.0 / N)
    m = jnp.maximum(l0, l1)
    x0 = jnp.exp(l0 - m)
    x1 = jnp.exp(l1 - m)
    b0 = x0 / (x0 + x1)
    b1 = x1 / (x0 + x1)
    z_ref[...] = (b0 * es_ref[c0, i].astype(jnp.float32)
                  + b1 * es_ref[c1, i].astype(jnp.float32))


def _body(h_ref, adjl_ref, adjr_ref, wt_ref, gp_ref, awt_ref, ab_ref, av_ref,
          z_ref, sf_ref, es_ref, cs_ref):
    k = pl.program_id(0)
    i = pl.program_id(1)

    @pl.when(k < 4)
    def _():
        @pl.when(i == 0)
        def _():
            sf_ref[...] = (
                jnp.dot(h_ref[...].astype(jnp.bfloat16),
                        wt_ref[0].astype(jnp.bfloat16),
                        preferred_element_type=jnp.float32)
                + gp_ref[0, 0, :][None, :]).astype(jnp.bfloat16)

        o = (jnp.dot(adjl_ref[0].astype(jnp.bfloat16), sf_ref[:N // 2],
                     preferred_element_type=jnp.float32)
             + jnp.dot(adjr_ref[0].astype(jnp.bfloat16), sf_ref[N // 2:],
                       preferred_element_type=jnp.float32)
             + gp_ref[0, 1, :][None, :])
        a = gp_ref[0, 2, :][None, :]
        e = jnp.where(o > 0, o, a * o).astype(jnp.bfloat16)
        es_ref[k, i] = e
        t = jnp.tanh(
            jnp.dot(e, awt_ref[0].astype(jnp.bfloat16),
                    preferred_element_type=jnp.float32)
            + ab_ref[k // 2, 0, :][None, :])
        part = jnp.sum(t, axis=0, keepdims=True)

        @pl.when(i == 0)
        def _():
            cs_ref[k] = part

        @pl.when(i > 0)
        def _():
            cs_ref[k] = cs_ref[k] + part

    @pl.when(k == 2)
    def _():
        _mix(cs_ref, av_ref, es_ref, z_ref, 0, i)

    @pl.when(k == 4)
    def _():
        _mix(cs_ref, av_ref, es_ref, z_ref, 1, i)


def kernel(h, mps, mp_edge, gcn_W, gcn_bfc, gcn_bias, gcn_a, att_W, att_b,
           att_v):
    del mp_edge
    gwt = jnp.swapaxes(gcn_W, 1, 2)
    awt = jnp.swapaxes(att_W, 1, 2)
    gp = jnp.stack(
        [gcn_bfc, gcn_bias, jnp.broadcast_to(gcn_a[:, None], (4, HID))],
        axis=1)
    ab = att_b[:, None, :]
    av = att_v[:, None, :]

    def zmap(k, i):
        blk = jnp.where(k < 2, 0,
                        jnp.where(k == 2, i,
                                  jnp.where(k == 3, NB - 1, NB + i)))
        return (blk, 0)

    z = pl.pallas_call(
        _body,
        grid=(5, NB),
        in_specs=[
            pl.BlockSpec((N, HID), lambda k, i: (jnp.minimum(k, 3) // 2, 0)),
            pl.BlockSpec(
                (1, BM, N // 2),
                lambda k, i: (jnp.minimum(k, 3),
                              jnp.where(k < 4, i, NB - 1), 0)),
            pl.BlockSpec(
                (1, BM, N // 2),
                lambda k, i: (jnp.minimum(k, 3),
                              jnp.where(k < 4, i, NB - 1), 1)),
            pl.BlockSpec((1, HID, HID), lambda k, i: (jnp.minimum(k, 3), 0, 0)),
            pl.BlockSpec((1, 3, HID), lambda k, i: (jnp.minimum(k, 3), 0, 0)),
            pl.BlockSpec((1, HID, HID),
                         lambda k, i: (jnp.minimum(k, 3) // 2, 0, 0)),
            pl.BlockSpec((2, 1, HID), lambda k, i: (0, 0, 0)),
            pl.BlockSpec((2, 1, HID), lambda k, i: (0, 0, 0)),
        ],
        out_specs=pl.BlockSpec((BM, HID), zmap),
        out_shape=jax.ShapeDtypeStruct((2 * N, HID), jnp.float32),
        scratch_shapes=[
            pltpu.VMEM((N, HID), jnp.bfloat16),
            pltpu.VMEM((4, NB, BM, HID), jnp.bfloat16),
            pltpu.VMEM((4, 1, HID), jnp.float32),
        ],
    )(h, mps, mps, gwt, gp, awt, ab, av)
    return z
